# baseline (speedup 1.0000x reference)
import functools

import jax
import jax.numpy as jnp
from jax import lax
from jax.experimental import pallas as pl
from jax.experimental.pallas import tpu as pltpu
from jax.experimental.pallas import tpu_sc as plsc

B, L = 4096, 200
V, FEA, K = 1000000, 64, 128

NC, NS = 2, 16
NW = NC * NS
BPW = B // NW
BPC = B // NC
TRASH = BPC
CA, CB, OB = 128, 80, 120


def _sc_pool(log, mask, emb_table):
    mesh = plsc.VectorSubcoreMesh(core_axis_name="c", subcore_axis_name="s")

    @functools.partial(
        pl.kernel,
        out_type=jax.ShapeDtypeStruct((B, FEA), jnp.float32),
        mesh=mesh,
        compiler_params=pltpu.CompilerParams(use_tc_tiling_on_sc=False),
        scratch_types=[
            pltpu.VMEM((BPW, L), jnp.int32),
            pltpu.VMEM((BPW,), jnp.int32),
            pltpu.VMEM((BPW, CA), jnp.int32),
            pltpu.VMEM((BPW, CB), jnp.int32),
            pltpu.VMEM((CA, FEA), jnp.float32),
            pltpu.VMEM((CB, FEA), jnp.float32),
            pltpu.VMEM_SHARED((BPC + 8, FEA), jnp.float32),
            pltpu.SemaphoreType.DMA,
            pltpu.SemaphoreType.DMA,
        ],
    )
    def k(table_h, log_h, mask_h, out_h,
          log_v, mask_v, dest_a, dest_b, buf_a, buf_b, acc, sem_a, sem_b):
        c = lax.axis_index("c")
        s = lax.axis_index("s")
        gbase = c * BPC + s * BPW
        dbase = s * BPW

        pltpu.sync_copy(log_h.at[pl.ds(gbase, BPW), :], log_v)
        pltpu.sync_copy(mask_h.at[pl.ds(gbase, BPW)], mask_v)

        def zrow(r, carry):
            for j in range(FEA // 16):
                buf_a[r, pl.ds(j * 16, 16)] = jnp.zeros((16,), jnp.float32)
            return carry
        lax.fori_loop(0, CA, zrow, 0)
        pltpu.sync_copy(buf_a.at[pl.ds(0, BPW), :], acc.at[pl.ds(dbase, BPW), :])

        iota = lax.iota(jnp.int32, 16)
        trash_vec = jnp.full((16,), TRASH, jnp.int32)

        def dfill(r, carry):
            mvec = mask_v[pl.ds((r >> 4) << 4, 16)]
            lens = mvec.at[jnp.full((16,), r & 15, jnp.int32)].get(
                mode="promise_in_bounds")
            dspl = jnp.full((16,), dbase + r, jnp.int32)
            for t in range(CA // 16):
                lvec = iota + (t * 16)
                dest_a[r, pl.ds(t * 16, 16)] = jnp.where(lvec < lens, dspl, trash_vec)
            for t in range(CB // 16):
                lvec = iota + (t * 16)
                ok = (lvec >= (CA - OB)) & ((lvec + OB) < lens)
                dest_b[r, pl.ds(t * 16, 16)] = jnp.where(ok, dspl, trash_vec)
            return carry
        lax.fori_loop(0, BPW, dfill, 0)

        def body(r, carry):
            cp_a = pltpu.async_copy(table_h.at[log_v.at[r, pl.ds(0, CA)]], buf_a, sem_a)
            cp_b = pltpu.async_copy(table_h.at[log_v.at[r, pl.ds(OB, CB)]], buf_b, sem_b)
            cp_a.wait()
            pltpu.sync_copy(buf_a, acc.at[dest_a.at[r]], add=True)
            cp_b.wait()
            pltpu.sync_copy(buf_b, acc.at[dest_b.at[r]], add=True)
            return carry
        lax.fori_loop(0, BPW, body, 0)

        pltpu.sync_copy(acc.at[pl.ds(dbase, BPW), :], out_h.at[pl.ds(gbase, BPW), :])

    return k(emb_table, log, mask)


def _tc_project(pooled, mask, W, b):
    BLK = 256

    def body(p_ref, m_ref, w_ref, b_ref, o_ref):
        lens = m_ref[...].astype(jnp.float32)
        x = p_ref[...] / lens
        o_ref[...] = (
            jnp.dot(x, w_ref[...], preferred_element_type=jnp.float32) + b_ref[...]
        )

    return pl.pallas_call(
        body,
        grid=(B // BLK,),
        in_specs=[
            pl.BlockSpec((BLK, FEA), lambda i: (i, 0)),
            pl.BlockSpec((BLK, 1), lambda i: (i, 0)),
            pl.BlockSpec((FEA, K), lambda i: (0, 0)),
            pl.BlockSpec((1, K), lambda i: (0, 0)),
        ],
        out_specs=pl.BlockSpec((BLK, K), lambda i: (i, 0)),
        out_shape=jax.ShapeDtypeStruct((B, K), jnp.float32),
    )(pooled, mask.reshape(B, 1), W, b.reshape(1, K))


def kernel(log, mask, emb_table, W, b):
    log = log.astype(jnp.int32)
    mask = mask.astype(jnp.int32)
    pooled = _sc_pool(log, mask, emb_table)
    return _tc_project(pooled, mask, W, b)
